# baseline (speedup 1.0000x reference)
import jax
import jax.numpy as jnp
from jax.experimental import pallas as pl
from jax.experimental.pallas import tpu as pltpu

_BSZ, _SEQ, _DIN, _DOUT, _E, _K, _R = 4, 2048, 768, 768, 64, 8, 16
_NBLK = 2
_SBLK = _SEQ // _NBLK


def _moelora_body(idx_ref, attn_ref, x_ref, w_ref, b_ref, ap_ref, bp_ref,
                  bias_ref, out_ref):
    bi = pl.program_id(0)
    a_parts = []
    b_parts = []
    bias_acc = b_ref[:]
    for k in range(_K):
        e = idx_ref[bi, k]
        w = attn_ref[bi, k]
        a_k = ap_ref[pl.ds(e, 1)].reshape(_DIN, _R)
        a_parts.append(a_k.T)
        b_parts.append(bp_ref[pl.ds(e, 1)].reshape(_R, _DOUT) * w)
        bias_acc = bias_acc + w * bias_ref[pl.ds(e, 1), :]
    acat_t = jnp.concatenate(a_parts, axis=0)
    bcat = jnp.concatenate(b_parts, axis=0)
    delta_t = jax.lax.dot_general(
        bcat.astype(jnp.bfloat16), acat_t.astype(jnp.bfloat16),
        (((0,), (0,)), ((), ())),
        preferred_element_type=jnp.float32)
    m_t = (w_ref[:] + delta_t).astype(jnp.bfloat16)
    out_ref[0] = jax.lax.dot_general(
        x_ref[0].astype(jnp.bfloat16), m_t,
        (((1,), (1,)), ((), ())),
        preferred_element_type=jnp.float32) + bias_acc


@jax.jit
def _run(x, attn, idx, w, b2, ap, bp, bias_pool):
    return pl.pallas_call(
        _moelora_body,
        grid=(_BSZ, _NBLK),
        in_specs=[
            pl.BlockSpec(memory_space=pltpu.SMEM),
            pl.BlockSpec(memory_space=pltpu.SMEM),
            pl.BlockSpec((1, _SBLK, _DIN), lambda i, j: (i, j, 0)),
            pl.BlockSpec((_DOUT, _DIN), lambda i, j: (0, 0)),
            pl.BlockSpec((1, _DOUT), lambda i, j: (0, 0)),
            pl.BlockSpec((_E, _DIN, _R), lambda i, j: (0, 0, 0)),
            pl.BlockSpec((_E, _R, _DOUT), lambda i, j: (0, 0, 0)),
            pl.BlockSpec((_E, _DOUT), lambda i, j: (0, 0)),
        ],
        out_specs=pl.BlockSpec((1, _SBLK, _DOUT), lambda i, j: (i, j, 0)),
        out_shape=jax.ShapeDtypeStruct((_BSZ, _SEQ, _DOUT), jnp.float32),
    )(idx, attn, x, w, b2, ap, bp, bias_pool)


def kernel(x, topk_attn, topk_idx, W, b, A_pool, B_pool, bias_pool):
    b2 = b.reshape(1, _DOUT)
    idx = topk_idx.astype(jnp.int32)
    return _run(x, topk_attn, idx, W, b2, A_pool, B_pool, bias_pool)


# speedup vs baseline: 1.0486x; 1.0486x over previous
import functools

import jax
import jax.numpy as jnp
from jax import lax
from jax.experimental import pallas as pl
from jax.experimental.pallas import tpu as pltpu
from jax.experimental.pallas import tpu_sc as plsc

_BSZ, _SEQ, _DIN, _DOUT, _E, _K, _R = 4, 2048, 768, 768, 64, 8, 16
_KR = _K * _R


def _sc_gather_body(rows_hbm, apt_hbm, bp_hbm, acat_hbm, bcat_hbm,
                    rows_v, a_rows, b_rows, sem):
    c = lax.axis_index("c")
    s = lax.axis_index("s")
    wid = s * 2 + c
    pltpu.sync_copy(rows_hbm.at[pl.ds(wid * _R, _R)], rows_v)
    pltpu.async_copy(apt_hbm.at[rows_v], a_rows, sem).wait()
    pltpu.sync_copy(a_rows, acat_hbm.at[pl.ds(wid * _R, _R)])
    pltpu.async_copy(bp_hbm.at[rows_v], b_rows, sem).wait()
    pltpu.sync_copy(b_rows, bcat_hbm.at[pl.ds(wid * _R, _R)])


_sc_gather = functools.partial(
    pl.kernel,
    mesh=plsc.VectorSubcoreMesh(core_axis_name="c", subcore_axis_name="s"),
    out_type=(
        jax.ShapeDtypeStruct((_BSZ * _KR, _DIN), jnp.float32),
        jax.ShapeDtypeStruct((_BSZ * _KR, _DOUT), jnp.float32),
    ),
    scratch_types=[
        pltpu.VMEM((_R,), jnp.int32),
        pltpu.VMEM((_R, _DIN), jnp.float32),
        pltpu.VMEM((_R, _DOUT), jnp.float32),
        pltpu.SemaphoreType.DMA,
    ],
)(_sc_gather_body)


def _moelora_body(idx_ref, attn_ref, x_ref, wt_ref, b_ref, acat_ref, bcat_ref,
                  bias_ref, out_ref):
    bi = pl.program_id(0)
    bias_acc = b_ref[:]
    scale_parts = []
    for k in range(_K):
        e = idx_ref[bi, k]
        w = attn_ref[bi, k]
        scale_parts.append(jnp.full((_R, 1), w, jnp.float32))
        bias_acc = bias_acc + w * bias_ref[pl.ds(e, 1), :]
    scale = jnp.concatenate(scale_parts, axis=0)
    acat_t = acat_ref[0]
    bcat = bcat_ref[0] * scale
    delta = jax.lax.dot_general(
        acat_t.astype(jnp.bfloat16), bcat.astype(jnp.bfloat16),
        (((0,), (0,)), ((), ())),
        preferred_element_type=jnp.float32)
    m = (wt_ref[:] + delta).astype(jnp.bfloat16)
    out_ref[0] = jnp.dot(x_ref[0].astype(jnp.bfloat16), m,
                         preferred_element_type=jnp.float32) + bias_acc


def _tc_run(x, attn, idx, wt, b2, acat, bcat, bias_pool):
    return pl.pallas_call(
        _moelora_body,
        grid=(_BSZ,),
        in_specs=[
            pl.BlockSpec(memory_space=pltpu.SMEM),
            pl.BlockSpec(memory_space=pltpu.SMEM),
            pl.BlockSpec((1, _SEQ, _DIN), lambda i: (i, 0, 0)),
            pl.BlockSpec((_DIN, _DOUT), lambda i: (0, 0)),
            pl.BlockSpec((1, _DOUT), lambda i: (0, 0)),
            pl.BlockSpec((1, _KR, _DIN), lambda i: (i, 0, 0)),
            pl.BlockSpec((1, _KR, _DOUT), lambda i: (i, 0, 0)),
            pl.BlockSpec((_E, _DOUT), lambda i: (0, 0)),
        ],
        out_specs=pl.BlockSpec((1, _SEQ, _DOUT), lambda i: (i, 0, 0)),
        out_shape=jax.ShapeDtypeStruct((_BSZ, _SEQ, _DOUT), jnp.float32),
    )(idx, attn, x, wt, b2, acat, bcat, bias_pool)


@jax.jit
def _run(x, attn, idx, wt, b2, apt2, bp2, bias_pool):
    rows = (idx.reshape(_BSZ * _K, 1) * _R
            + jnp.arange(_R, dtype=jnp.int32)).reshape(_BSZ * _K * _R)
    acat2, bcat2 = _sc_gather(rows, apt2, bp2)
    acat = acat2.reshape(_BSZ, _KR, _DIN)
    bcat = bcat2.reshape(_BSZ, _KR, _DOUT)
    return _tc_run(x, attn, idx, wt, b2, acat, bcat, bias_pool)


def kernel(x, topk_attn, topk_idx, W, b, A_pool, B_pool, bias_pool):
    wt = W.T
    apt2 = A_pool.transpose(0, 2, 1).reshape(_E * _R, _DIN)
    bp2 = B_pool.reshape(_E * _R, _DOUT)
    b2 = b.reshape(1, _DOUT)
    idx = topk_idx.astype(jnp.int32)
    return _run(x, topk_attn, idx, wt, b2, apt2, bp2, bias_pool)


# speedup vs baseline: 1.9663x; 1.8752x over previous
import jax
import jax.numpy as jnp
from jax.experimental import pallas as pl
from jax.experimental.pallas import tpu as pltpu

_BSZ, _SEQ, _DIN, _DOUT, _E, _K, _R = 4, 2048, 768, 768, 64, 8, 16


def _moelora_body(idx_ref, attn_ref, x_ref, wt_ref, b_ref, apt_ref, bp_ref,
                  bias_ref, out_ref):
    bi = pl.program_id(0)
    a_parts = []
    b_parts = []
    bias_acc = b_ref[:]
    for k in range(_K):
        e = idx_ref[bi, k]
        w = attn_ref[bi, k]
        a_parts.append(apt_ref[pl.ds(e, 1)].reshape(_R, _DIN))
        b_parts.append(bp_ref[pl.ds(e, 1)].reshape(_R, _DOUT) * w)
        bias_acc = bias_acc + w * bias_ref[pl.ds(e, 1), :]
    acat_t = jnp.concatenate(a_parts, axis=0)
    bcat = jnp.concatenate(b_parts, axis=0)
    delta_t = jax.lax.dot_general(
        bcat.astype(jnp.bfloat16), acat_t.astype(jnp.bfloat16),
        (((0,), (0,)), ((), ())),
        preferred_element_type=jnp.float32)
    m_t = (wt_ref[:] + delta_t).astype(jnp.bfloat16)
    out_ref[0] = jax.lax.dot_general(
        x_ref[0].astype(jnp.bfloat16), m_t,
        (((1,), (1,)), ((), ())),
        preferred_element_type=jnp.float32) + bias_acc


@jax.jit
def _run(x, attn, idx, wt, b2, apt, bp, bias_pool):
    return pl.pallas_call(
        _moelora_body,
        grid=(_BSZ,),
        in_specs=[
            pl.BlockSpec(memory_space=pltpu.SMEM),
            pl.BlockSpec(memory_space=pltpu.SMEM),
            pl.BlockSpec((1, _SEQ, _DIN), lambda i: (i, 0, 0)),
            pl.BlockSpec((_DOUT, _DIN), lambda i: (0, 0)),
            pl.BlockSpec((1, _DOUT), lambda i: (0, 0)),
            pl.BlockSpec((_E, _R, _DIN), lambda i: (0, 0, 0)),
            pl.BlockSpec((_E, _R, _DOUT), lambda i: (0, 0, 0)),
            pl.BlockSpec((_E, _DOUT), lambda i: (0, 0)),
        ],
        out_specs=pl.BlockSpec((1, _SEQ, _DOUT), lambda i: (i, 0, 0)),
        out_shape=jax.ShapeDtypeStruct((_BSZ, _SEQ, _DOUT), jnp.float32),
    )(idx, attn, x, wt, b2, apt, bp, bias_pool)


def kernel(x, topk_attn, topk_idx, W, b, A_pool, B_pool, bias_pool):
    apt = A_pool.transpose(0, 2, 1)
    b2 = b.reshape(1, _DOUT)
    idx = topk_idx.astype(jnp.int32)
    return _run(x, topk_attn, idx, W, b2, apt, B_pool, bias_pool)
